# baseline (speedup 1.0000x reference)
import functools

import jax
import jax.numpy as jnp
from jax import lax
from jax.experimental import pallas as pl
from jax.experimental.pallas import tpu as pltpu
from jax.experimental.pallas import tpu_sc as plsc

_B = 4096
_L = 200
_NITEMS = 100000
_NUSERS = 10000
_NC = 2
_NS = 16
_NW = _NC * _NS
_PER_W = _B * _L // _NW
_CHUNK = 128
_SUPER = 3200
_NSUPER = _PER_W // _SUPER
_CPS = _SUPER // _CHUNK
_UPW = _B // _NW


def _sc_body(logf, t1f, t2f, ridxf, mt, wt, rt, om, ow, orr,
             cbuf, t1buf, t2buf, rbuf, midx, widx, ridx,
             mrows, wrows, rrows, sem):
    wid = lax.axis_index("s") * _NC + lax.axis_index("c")
    base = wid * _PER_W

    def super_body(s, _):
        soff = base + s * _SUPER
        pltpu.sync_copy(logf.at[pl.ds(soff, _SUPER)], cbuf)
        pltpu.sync_copy(t1f.at[pl.ds(soff, _SUPER)], t1buf)
        pltpu.sync_copy(t2f.at[pl.ds(soff, _SUPER)], t2buf)
        pltpu.sync_copy(ridxf.at[pl.ds(soff, _SUPER)], rbuf)

        def chunk_body(jj, _):
            coff = jj * _CHUNK
            for v in range(_CHUNK // 16):
                sl = pl.ds(coff + v * 16, 16)
                dsl = pl.ds(v * 16, 16)
                c = cbuf[sl]
                t1 = t1buf[sl]
                t2 = t2buf[sl]
                midx[dsl] = c * 24 + t1
                widx[dsl] = c * 105 + t2
                ridx[dsl] = rbuf[sl]
            cm = pltpu.async_copy(mt.at[midx], mrows, sem)
            cw = pltpu.async_copy(wt.at[widx], wrows, sem)
            cr = pltpu.async_copy(rt.at[ridx], rrows, sem)
            cm.wait()
            cw.wait()
            cr.wait()
            f0 = soff + coff
            pltpu.sync_copy(mrows, om.at[pl.ds(f0, _CHUNK), :])
            pltpu.sync_copy(wrows, ow.at[pl.ds(f0, _CHUNK), :])
            pltpu.sync_copy(rrows, orr.at[pl.ds(f0, _CHUNK), :])
            return 0

        lax.fori_loop(0, _CPS, chunk_body, 0)
        return 0

    lax.fori_loop(0, _NSUPER, super_body, 0)


def _sc_gather(logf, t1f, t2f, ridxf, mt, wt, rt):
    mesh = plsc.VectorSubcoreMesh(
        core_axis_name="c", subcore_axis_name="s",
        num_cores=_NC, num_subcores=_NS)
    return pl.kernel(
        _sc_body,
        out_type=(
            jax.ShapeDtypeStruct((_B * _L, 16), jnp.float32),
            jax.ShapeDtypeStruct((_B * _L, 8), jnp.float32),
            jax.ShapeDtypeStruct((_B * _L, 8), jnp.float32),
        ),
        mesh=mesh,
        compiler_params=pltpu.CompilerParams(use_tc_tiling_on_sc=False),
        scratch_types=[
            pltpu.VMEM((_SUPER,), jnp.int32),
            pltpu.VMEM((_SUPER,), jnp.int32),
            pltpu.VMEM((_SUPER,), jnp.int32),
            pltpu.VMEM((_SUPER,), jnp.int32),
            pltpu.VMEM((_CHUNK,), jnp.int32),
            pltpu.VMEM((_CHUNK,), jnp.int32),
            pltpu.VMEM((_CHUNK,), jnp.int32),
            pltpu.VMEM((_CHUNK, 16), jnp.float32),
            pltpu.VMEM((_CHUNK, 8), jnp.float32),
            pltpu.VMEM((_CHUNK, 8), jnp.float32),
            pltpu.SemaphoreType.DMA,
        ],
    )(logf, t1f, t2f, ridxf, mt, wt, rt)


def kernel(log_seqs, time1_seqs, time2_seqs, user, month_pop, week_pop, week_eval_pop):
    mt = jnp.pad(month_pop.T.reshape(_NITEMS, 24, 12),
                 ((1, 0), (0, 0), (0, 4))).reshape(-1, 16)
    wt = jnp.pad(week_pop.T.reshape(_NITEMS, 104, 6),
                 ((1, 0), (1, 0), (0, 2))).reshape(-1, 8)
    rt = jnp.pad(week_eval_pop.reshape(_NUSERS, 6, _L).transpose(0, 2, 1),
                 ((0, 0), (0, 0), (0, 2))).reshape(-1, 8)
    ueff = ((user.astype(jnp.int32) + (_NUSERS - 1)) % _NUSERS) * _L
    ridxf = (ueff[:, None] + jnp.arange(_L, dtype=jnp.int32)[None, :]).reshape(-1)
    logf = log_seqs.reshape(-1).astype(jnp.int32)
    t1f = time1_seqs.reshape(-1).astype(jnp.int32)
    t2f = time2_seqs.reshape(-1).astype(jnp.int32)
    om, ow, orr = _sc_gather(logf, t1f, t2f, ridxf, mt, wt, rt)
    out = jnp.concatenate([om[:, :12], ow[:, :6], orr[:, :6]], axis=1)
    return out.reshape(_B, _L, 24)


# speedup vs baseline: 1.5400x; 1.5400x over previous
import jax
import jax.numpy as jnp
import numpy as np
from jax import lax
from jax.experimental import pallas as pl
from jax.experimental.pallas import tpu as pltpu
from jax.experimental.pallas import tpu_sc as plsc

_B = 4096
_L = 200
_NITEMS = 100000
_NUSERS = 10000
_NC = 2
_NS = 16
_NW = _NC * _NS
_PER_W = _B * _L // _NW
_CHUNK = 128
_SUPER = 3200
_NSUPER = _PER_W // _SUPER
_CPS = _SUPER // _CHUNK
_UPW = _B // _NW
_MWROWS = (_NITEMS + 1) * 24 // 8
_WWROWS = ((_NITEMS + 1) * 105 * 8 + 56) // 128
_RWROWS = _NUSERS * 16


def _sc_body(logf, t1f, t2f, ridxf, colpat, mt1, wt1, rt1, out,
             cbuf, t1buf, t2buf, rbuf, colbuf, midx, widx,
             moff, woff, rwidx, mbig, wbig, rbig, obuf, sem):
    wid = lax.axis_index("s") * _NC + lax.axis_index("c")
    base = wid * _PER_W
    iota = lax.iota(jnp.int32, 16)
    iota24 = iota * 24
    pltpu.sync_copy(colpat, colbuf)

    def super_body(s, _):
        soff = base + s * _SUPER
        pltpu.sync_copy(logf.at[pl.ds(soff, _SUPER)], cbuf)
        pltpu.sync_copy(t1f.at[pl.ds(soff, _SUPER)], t1buf)
        pltpu.sync_copy(t2f.at[pl.ds(soff, _SUPER)], t2buf)
        pltpu.sync_copy(ridxf.at[pl.ds(soff, _SUPER)], rbuf)

        def chunk_body(jj, _):
            coff = jj * _CHUNK
            for v in range(_CHUNK // 16):
                sl = pl.ds(coff + v * 16, 16)
                dsl = pl.ds(v * 16, 16)
                m = cbuf[sl] * 24 + t1buf[sl]
                w = cbuf[sl] * 105 + t2buf[sl]
                midx[dsl] = lax.shift_right_logical(m, 3)
                moff[dsl] = (m & 7) * 16
                widx[dsl] = lax.shift_right_logical(w, 4)
                woff[dsl] = (w & 15) * 8
                rwidx[dsl] = rbuf[sl]
            cm = pltpu.async_copy(mt1.at[midx], mbig, sem)
            cw = pltpu.async_copy(wt1.at[widx], wbig, sem)
            cr = pltpu.async_copy(rt1.at[rwidx], rbig, sem)
            cm.wait()
            cw.wait()
            cr.wait()
            for v in range(_CHUNK // 16):
                dsl = pl.ds(v * 16, 16)
                rows = iota + (v * 16)
                mo = moff[dsl]
                wo = woff[dsl]
                rc = colbuf[pl.ds(coff + v * 16, 16)]
                od = iota24 + (v * 384)
                for k in range(12):
                    val = plsc.load_gather(mbig, [rows, mo + k])
                    plsc.store_scatter(obuf, [od + k], val)
                for k in range(6):
                    val = plsc.load_gather(wbig, [rows, wo + k])
                    plsc.store_scatter(obuf, [od + (12 + k)], val)
                for k in range(6):
                    val = plsc.load_gather(rbig, [rows, rc + k])
                    plsc.store_scatter(obuf, [od + (18 + k)], val)
            pltpu.sync_copy(obuf, out.at[pl.ds((soff + coff) * 24,
                                               _CHUNK * 24)])
            return 0

        lax.fori_loop(0, _CPS, chunk_body, 0)
        return 0

    lax.fori_loop(0, _NSUPER, super_body, 0)


def _sc_gather(logf, t1f, t2f, ridxf, colpat, mt1, wt1, rt1):
    mesh = plsc.VectorSubcoreMesh(
        core_axis_name="c", subcore_axis_name="s",
        num_cores=_NC, num_subcores=_NS)
    return pl.kernel(
        _sc_body,
        out_type=jax.ShapeDtypeStruct((_B * _L * 24,), jnp.float32),
        mesh=mesh,
        compiler_params=pltpu.CompilerParams(use_tc_tiling_on_sc=False, needs_layout_passes=False),
        scratch_types=[
            pltpu.VMEM((_SUPER,), jnp.int32),
            pltpu.VMEM((_SUPER,), jnp.int32),
            pltpu.VMEM((_SUPER,), jnp.int32),
            pltpu.VMEM((_SUPER,), jnp.int32),
            pltpu.VMEM((_SUPER,), jnp.int32),
            pltpu.VMEM((_CHUNK,), jnp.int32),
            pltpu.VMEM((_CHUNK,), jnp.int32),
            pltpu.VMEM((_CHUNK,), jnp.int32),
            pltpu.VMEM((_CHUNK,), jnp.int32),
            pltpu.VMEM((_CHUNK,), jnp.int32),
            pltpu.VMEM((_CHUNK, 128), jnp.float32),
            pltpu.VMEM((_CHUNK, 128), jnp.float32),
            pltpu.VMEM((_CHUNK, 128), jnp.float32),
            pltpu.VMEM((_CHUNK * 24,), jnp.float32),
            pltpu.SemaphoreType.DMA,
        ],
    )(logf, t1f, t2f, ridxf, colpat, mt1, wt1, rt1)


def kernel(log_seqs, time1_seqs, time2_seqs, user, month_pop, week_pop, week_eval_pop):
    mt1 = jnp.pad(month_pop.T.reshape(_NITEMS, 24, 12),
                  ((1, 0), (0, 0), (0, 4))).reshape(_MWROWS, 128)
    wt1 = jnp.pad(jnp.pad(week_pop.T.reshape(_NITEMS, 104, 6),
                          ((1, 0), (1, 0), (0, 2))).reshape(-1),
                  (0, 56)).reshape(_WWROWS, 128)
    rt1 = jnp.pad(week_eval_pop.reshape(_NUSERS, 6, _L).transpose(0, 2, 1),
                  ((0, 0), (0, 56), (0, 2))).reshape(_RWROWS, 128)
    ueff16 = (((user.astype(jnp.int32) + (_NUSERS - 1)) % _NUSERS) * 16)
    lrow = (jnp.arange(_L, dtype=jnp.int32) // 16)[None, :]
    ridxf = (ueff16[:, None] + lrow).reshape(-1)
    p = np.arange(_SUPER)
    colpat = jnp.asarray(((p % _L) % 16) * 8, dtype=jnp.int32)
    logf = log_seqs.reshape(-1).astype(jnp.int32)
    t1f = time1_seqs.reshape(-1).astype(jnp.int32)
    t2f = time2_seqs.reshape(-1).astype(jnp.int32)
    out = _sc_gather(logf, t1f, t2f, ridxf, colpat, mt1, wt1, rt1)
    return out.reshape(_B, _L, 24)
